# baseline (speedup 1.0000x reference)
import functools
import math

import jax
import jax.numpy as jnp
from jax.experimental import pallas as pl
from jax.experimental.pallas import tpu as pltpu

NH = 16
GH, GW = 64, 32
S = GH * GW
DH = 128
D = NH * DH
BQ = 256
NT = S // BQ
NEG = -1e9
SCALE = math.log2(math.e) / math.sqrt(DH)


def _window_bias():
    iq = jnp.arange(BQ)[:, None]
    ik = jnp.arange(3 * BQ)[None, :] - BQ
    dr = (ik >> 5) - (iq >> 5)
    dc = (ik & 31) - (iq & 31)
    mask = (dr >= -3) & (dr <= 2) & (dc >= -3) & (dc <= 2)
    return jnp.where(mask, 0.0, NEG).astype(jnp.float32)


def _fused_kernel(xa_ref, wqkv_ref, wo_ref, bias_ref, o_ref, ring_ref,
                  ao_ref):
    i = pl.program_id(0)

    @pl.when(i < NT)
    def _qkv_tile():
        cw = 512
        for c in range(0, 3 * D, cw):
            out = jnp.dot(xa_ref[...], wqkv_ref[:, c:c + cw],
                          preferred_element_type=jnp.float32)
            if c + cw <= D:
                out = out * jnp.float32(SCALE)
            ring_ref[i & 3, :, c:c + cw] = out.astype(jnp.bfloat16)

    @pl.when(i >= 2)
    def _attention():
        t = i - 2
        slot = [jnp.clip(t - 1 + j, 0, NT - 1) & 3 for j in range(3)]
        q_slot = t & 3
        for h in range(NH):
            qcols = slice(h * DH, (h + 1) * DH)
            kcols = slice(D + h * DH, D + (h + 1) * DH)
            vcols = slice(2 * D + h * DH, 2 * D + (h + 1) * DH)
            qh = ring_ref[q_slot, :, qcols]
            sc = []
            for j in range(3):
                kj = ring_ref[slot[j], :, kcols]
                raw = jax.lax.dot_general(
                    qh, kj, (((1,), (1,)), ((), ())),
                    preferred_element_type=jnp.float32)
                valid = jnp.logical_and(t - 1 + j >= 0, t - 1 + j < NT)
                bias_j = bias_ref[:, j * BQ:(j + 1) * BQ]
                sc.append(jnp.where(valid, raw + bias_j, NEG))
            m = jnp.maximum(
                jnp.maximum(jnp.max(sc[0], axis=1, keepdims=True),
                            jnp.max(sc[1], axis=1, keepdims=True)),
                jnp.max(sc[2], axis=1, keepdims=True))
            e = [jnp.exp2(x - m) for x in sc]
            s = (jnp.sum(e[0], axis=1, keepdims=True)
                 + jnp.sum(e[1], axis=1, keepdims=True)
                 + jnp.sum(e[2], axis=1, keepdims=True))
            o = sum(jnp.dot(e[j].astype(jnp.bfloat16),
                            ring_ref[slot[j], :, vcols],
                            preferred_element_type=jnp.float32)
                    for j in range(3))
            ao_ref[:, qcols] = (o * (1.0 / s)).astype(jnp.bfloat16)
        o_ref[...] = jnp.dot(ao_ref[...], wo_ref[...],
                             preferred_element_type=jnp.float32)


def kernel(x, Wqkv, Wo):
    B, S_, Dx = x.shape
    x2 = x.reshape(S_, Dx)
    clip = lambda v: jnp.clip(v, 0, NT - 1)
    xa_spec = pl.BlockSpec((BQ, Dx), lambda i: (clip(i), 0))
    wqkv_spec = pl.BlockSpec((Dx, 3 * D), lambda i: (0, 0))
    wo_spec = pl.BlockSpec((D, D), lambda i: (0, 0))
    bias_spec = pl.BlockSpec((BQ, 3 * BQ), lambda i: (0, 0))
    out = pl.pallas_call(
        _fused_kernel,
        grid=(NT + 2,),
        in_specs=[xa_spec, wqkv_spec, wo_spec, bias_spec],
        out_specs=pl.BlockSpec((BQ, D), lambda i: (clip(i - 2), 0)),
        out_shape=jax.ShapeDtypeStruct((S, D), jnp.float32),
        scratch_shapes=[pltpu.VMEM((4, BQ, 3 * D), jnp.bfloat16),
                        pltpu.VMEM((BQ, D), jnp.bfloat16)],
        compiler_params=pltpu.CompilerParams(
            dimension_semantics=("arbitrary",),
            vmem_limit_bytes=67_000_000),
    )(x2, Wqkv.astype(jnp.bfloat16), Wo.astype(jnp.bfloat16), _window_bias())
    return out.reshape(B, S_, Dx)


# speedup vs baseline: 1.1683x; 1.1683x over previous
import functools
import math

import jax
import jax.numpy as jnp
from jax.experimental import pallas as pl
from jax.experimental.pallas import tpu as pltpu

NH = 16
GH, GW = 64, 32
S = GH * GW
DH = 128
BQ = 256
NT = S // BQ
NEG = -1e9


def _qkv_matmul_kernel(a_ref, b_ref, o_ref):
    j = pl.program_id(0)
    out = jnp.dot(a_ref[...], b_ref[...], preferred_element_type=jnp.float32)
    col = j * QKV_BN + jax.lax.broadcasted_iota(jnp.int32, (1, QKV_BN), 1)
    scale = jnp.where(col < NH * DH,
                      jnp.float32(math.log2(math.e) / math.sqrt(DH)),
                      jnp.float32(1.0))
    o_ref[...] = (out * scale).astype(o_ref.dtype)


QKV_BN = 768


def _qkv_matmul(a, b):
    M, K = a.shape
    _, N = b.shape
    return pl.pallas_call(
        _qkv_matmul_kernel,
        grid=(N // QKV_BN,),
        in_specs=[pl.BlockSpec((M, K), lambda j: (0, 0)),
                  pl.BlockSpec((K, QKV_BN), lambda j: (0, j))],
        out_specs=pl.BlockSpec((M, QKV_BN), lambda j: (0, j)),
        out_shape=jax.ShapeDtypeStruct((M, N), jnp.bfloat16),
        compiler_params=pltpu.CompilerParams(
            dimension_semantics=("arbitrary",)),
    )(a, b)


def _window_bias():
    iq = jnp.arange(BQ)[:, None]
    ik = jnp.arange(3 * BQ)[None, :] - BQ
    dr = (ik >> 5) - (iq >> 5)
    dc = (ik & 31) - (iq & 31)
    mask = (dr >= -3) & (dr <= 2) & (dc >= -3) & (dc <= 2)
    return jnp.where(mask, 0.0, NEG).astype(jnp.float32)


def _attn_kernel(q_ref, k0_ref, k1_ref, k2_ref, v0_ref, v1_ref, v2_ref,
                 wo_ref, bias_ref, o_ref, ao_ref, wob_ref):
    t = pl.program_id(0)
    k_refs = (k0_ref, k1_ref, k2_ref)
    v_refs = (v0_ref, v1_ref, v2_ref)

    @pl.when(t == 0)
    def _pack_wo():
        wob_ref[...] = wo_ref[...].astype(jnp.bfloat16)

    for h in range(NH):
        cols = slice(h * DH, (h + 1) * DH)
        qh = q_ref[:, cols]
        sc = []
        for j in range(3):
            raw = jax.lax.dot_general(
                qh, k_refs[j][:, cols], (((1,), (1,)), ((), ())),
                preferred_element_type=jnp.float32)
            valid = jnp.logical_and(t - 1 + j >= 0, t - 1 + j < NT)
            bias_j = bias_ref[:, j * BQ:(j + 1) * BQ]
            sc.append(jnp.where(valid, raw + bias_j, NEG))
        m = jnp.maximum(
            jnp.maximum(jnp.max(sc[0], axis=1, keepdims=True),
                        jnp.max(sc[1], axis=1, keepdims=True)),
            jnp.max(sc[2], axis=1, keepdims=True))
        e = [jnp.exp2(x - m) for x in sc]
        s = (jnp.sum(e[0], axis=1, keepdims=True)
             + jnp.sum(e[1], axis=1, keepdims=True)
             + jnp.sum(e[2], axis=1, keepdims=True))
        o = sum(jnp.dot(e[j].astype(jnp.bfloat16), v_refs[j][:, cols],
                        preferred_element_type=jnp.float32)
                for j in range(3))
        ao_ref[:, cols] = (o * (1.0 / s)).astype(jnp.bfloat16)
    o_ref[...] = jnp.dot(ao_ref[...], wob_ref[...],
                         preferred_element_type=jnp.float32)


def _banded_attention(qkv, wo):
    D = NH * DH
    clip = lambda i: jnp.clip(i, 0, NT - 1)
    q_spec = pl.BlockSpec((BQ, D), lambda t: (t, 0))
    k_specs = [pl.BlockSpec((BQ, D),
                            functools.partial(
                                lambda j, t: (clip(t - 1 + j), 1), j))
               for j in range(3)]
    v_specs = [pl.BlockSpec((BQ, D),
                            functools.partial(
                                lambda j, t: (clip(t - 1 + j), 2), j))
               for j in range(3)]
    wo_spec = pl.BlockSpec((D, D), lambda t: (0, 0))
    bias_spec = pl.BlockSpec((BQ, 3 * BQ), lambda t: (0, 0))
    return pl.pallas_call(
        _attn_kernel,
        grid=(NT,),
        in_specs=[q_spec] + k_specs + v_specs + [wo_spec, bias_spec],
        out_specs=pl.BlockSpec((BQ, D), lambda t: (t, 0)),
        out_shape=jax.ShapeDtypeStruct((S, D), jnp.float32),
        scratch_shapes=[pltpu.VMEM((BQ, D), jnp.bfloat16),
                        pltpu.VMEM((D, D), jnp.bfloat16)],
        compiler_params=pltpu.CompilerParams(
            dimension_semantics=("arbitrary",)),
    )(qkv, qkv, qkv, qkv, qkv, qkv, qkv, wo, _window_bias())


def kernel(x, Wqkv, Wo):
    B, S_, D = x.shape
    x2 = x.reshape(S_, D)
    qkv = _qkv_matmul(x2, Wqkv)
    out = _banded_attention(qkv, Wo)
    return out.reshape(B, S_, D)
